# baseline (speedup 1.0000x reference)
import functools

import jax
import jax.numpy as jnp
from jax import lax
from jax.experimental import pallas as pl
from jax.experimental.pallas import tpu as pltpu
from jax.experimental.pallas import tpu_sc as plsc

N = 8192
D_MODEL = 1024
OUT_DIM = 256
E = 8
G = 64
BN = 512
NB = N // BN

SC_NC = 2
SC_NS = 16
SC_NW = SC_NC * SC_NS
SC_L = 128
B_PER_W = N // SC_NW
CH = 128


def _sc_route_kernel(table_hbm, idx_hbm, out_hbm, idx_v, rows_v, sem):
    wid = lax.axis_index("s") * SC_NC + lax.axis_index("c")
    base = wid * B_PER_W
    for c in range(B_PER_W // CH):
        off = base + c * CH
        pltpu.sync_copy(idx_hbm.at[pl.ds(off, CH)], idx_v)
        pltpu.async_copy(table_hbm.at[idx_v], rows_v, sem).wait()
        pltpu.sync_copy(rows_v, out_hbm.at[pl.ds(off, CH)])


_sc_route = functools.partial(
    pl.kernel,
    mesh=plsc.VectorSubcoreMesh(core_axis_name="c", subcore_axis_name="s"),
    out_type=jax.ShapeDtypeStruct((N, SC_L), jnp.int32),
    scratch_types=[
        pltpu.VMEM((CH,), jnp.int32),
        pltpu.VMEM((CH, SC_L), jnp.int32),
        pltpu.SemaphoreType.DMA,
    ],
)(_sc_route_kernel)


def _moe_block_kernel(bits_ref, eat_ref, emb_ref, W_ref, b_ref,
                      out_hbm, y0, y1, sem0, sem1):
    i = pl.program_id(0)
    bits = bits_ref[i]
    e_atom = eat_ref[:, :1]
    x = emb_ref[...].astype(jnp.bfloat16)

    def run(y_ref, sem):
        @pl.when(i >= 2)
        def _():
            pltpu.make_async_copy(
                y_ref, out_hbm.at[:, pl.ds((i - 2) * BN, BN), :], sem
            ).wait()

        for e in range(E):
            present = ((bits >> e) & 1) == 1

            @pl.when(present)
            def _(e=e):
                mask = e_atom == e
                y = jnp.dot(x, W_ref[e].astype(jnp.bfloat16),
                            preferred_element_type=jnp.float32)
                y = y + b_ref[pl.ds(e, 1), :]
                y_ref[e] = jnp.where(mask, y, 0.0)

            @pl.when(jnp.logical_not(present))
            def _(e=e):
                y_ref[e] = jnp.zeros((BN, OUT_DIM), jnp.float32)

        pltpu.make_async_copy(
            y_ref, out_hbm.at[:, pl.ds(i * BN, BN), :], sem
        ).start()

    @pl.when(i % 2 == 0)
    def _():
        run(y0, sem0)

    @pl.when(i % 2 == 1)
    def _():
        run(y1, sem1)

    @pl.when(i == NB - 1)
    def _():
        pltpu.make_async_copy(
            y0, out_hbm.at[:, pl.ds((NB - 2) * BN, BN), :], sem0
        ).wait()
        pltpu.make_async_copy(
            y1, out_hbm.at[:, pl.ds((NB - 1) * BN, BN), :], sem1
        ).wait()


def kernel(emb, W, b, batch_idx, dataset_idx):
    bi = batch_idx.astype(jnp.int32)
    d32 = dataset_idx.astype(jnp.int32)
    table = jnp.broadcast_to(d32[:, None], (G, SC_L))
    eat = _sc_route(table, bi)
    br = bi.reshape(NB, BN)
    g_lo = br[:, 0]
    g_hi = br[:, BN - 1]
    g_ar = jnp.arange(G, dtype=jnp.int32)
    rng = (g_ar[None, :] >= g_lo[:, None]) & (g_ar[None, :] <= g_hi[:, None])
    presence = jnp.any(rng[:, :, None]
                       & (d32[None, :, None] == jnp.arange(E)[None, None, :]),
                       axis=1)
    bits = jnp.sum(presence.astype(jnp.int32)
                   << jnp.arange(E, dtype=jnp.int32)[None, :], axis=1)

    out = pl.pallas_call(
        _moe_block_kernel,
        grid_spec=pltpu.PrefetchScalarGridSpec(
            num_scalar_prefetch=1,
            grid=(NB,),
            in_specs=[
                pl.BlockSpec((BN, SC_L), lambda i, bits_ref: (i, 0)),
                pl.BlockSpec((BN, D_MODEL), lambda i, bits_ref: (i, 0)),
                pl.BlockSpec((E, D_MODEL, OUT_DIM),
                             lambda i, bits_ref: (0, 0, 0)),
                pl.BlockSpec((E, OUT_DIM), lambda i, bits_ref: (0, 0)),
            ],
            out_specs=pl.BlockSpec(memory_space=pl.MemorySpace.ANY),
            scratch_shapes=[
                pltpu.VMEM((E, BN, OUT_DIM), jnp.float32),
                pltpu.VMEM((E, BN, OUT_DIM), jnp.float32),
                pltpu.SemaphoreType.DMA,
                pltpu.SemaphoreType.DMA,
            ],
        ),
        out_shape=jax.ShapeDtypeStruct((E, N, OUT_DIM), jnp.float32),
        compiler_params=pltpu.CompilerParams(
            dimension_semantics=("arbitrary",),
        ),
    )(bits, eat, emb, W, b)
    return out


# speedup vs baseline: 1.5025x; 1.5025x over previous
import jax
import jax.numpy as jnp
from jax.experimental import pallas as pl
from jax.experimental.pallas import tpu as pltpu

N = 8192
D_MODEL = 1024
OUT_DIM = 256
E = 8
G = 64
BN = 512
NB = N // BN


def _moe_block_kernel(bits_ref, bidx_ref, didx_ref, emb_ref, W_ref, b_ref,
                      out_hbm, y0, y1, sem0, sem1):
    i = pl.program_id(0)
    bits = bits_ref[i]
    bidx = bidx_ref[0]
    g_iota = jax.lax.broadcasted_iota(jnp.int32, (BN, G), 1)
    onehot = bidx == g_iota
    didx = didx_ref[...]
    e_atom = jnp.sum(jnp.where(onehot, didx, 0), axis=1,
                     keepdims=True)
    x = emb_ref[...].astype(jnp.bfloat16)

    def run(y_ref, sem):
        @pl.when(i >= 2)
        def _():
            pltpu.make_async_copy(
                y_ref, out_hbm.at[:, pl.ds((i - 2) * BN, BN), :], sem
            ).wait()

        for e in range(E):
            present = ((bits >> e) & 1) == 1

            @pl.when(present)
            def _(e=e):
                mask = e_atom == e
                y = jnp.dot(x, W_ref[e].astype(jnp.bfloat16),
                            preferred_element_type=jnp.float32)
                y = y + b_ref[pl.ds(e, 1), :]
                y_ref[e] = jnp.where(mask, y, 0.0)

            @pl.when(jnp.logical_not(present))
            def _(e=e):
                y_ref[e] = jnp.zeros((BN, OUT_DIM), jnp.float32)

        pltpu.make_async_copy(
            y_ref, out_hbm.at[:, pl.ds(i * BN, BN), :], sem
        ).start()

    @pl.when(i % 2 == 0)
    def _():
        run(y0, sem0)

    @pl.when(i % 2 == 1)
    def _():
        run(y1, sem1)

    @pl.when(i == NB - 1)
    def _():
        pltpu.make_async_copy(
            y0, out_hbm.at[:, pl.ds((NB - 2) * BN, BN), :], sem0
        ).wait()
        pltpu.make_async_copy(
            y1, out_hbm.at[:, pl.ds((NB - 1) * BN, BN), :], sem1
        ).wait()


def kernel(emb, W, b, batch_idx, dataset_idx):
    bi = batch_idx.astype(jnp.int32)
    bidx = bi.reshape(NB, BN, 1)
    didx = dataset_idx.astype(jnp.int32).reshape(1, G)
    br = bi.reshape(NB, BN)
    g_lo = br[:, 0]
    g_hi = br[:, BN - 1]
    g_ar = jnp.arange(G, dtype=jnp.int32)
    rng = (g_ar[None, :] >= g_lo[:, None]) & (g_ar[None, :] <= g_hi[:, None])
    d32 = dataset_idx.astype(jnp.int32)
    presence = jnp.any(rng[:, :, None]
                       & (d32[None, :, None] == jnp.arange(E)[None, None, :]),
                       axis=1)
    bits = jnp.sum(presence.astype(jnp.int32)
                   << jnp.arange(E, dtype=jnp.int32)[None, :], axis=1)

    out = pl.pallas_call(
        _moe_block_kernel,
        grid_spec=pltpu.PrefetchScalarGridSpec(
            num_scalar_prefetch=1,
            grid=(NB,),
            in_specs=[
                pl.BlockSpec((1, BN, 1), lambda i, bits_ref: (i, 0, 0)),
                pl.BlockSpec((1, G), lambda i, bits_ref: (0, 0)),
                pl.BlockSpec((BN, D_MODEL), lambda i, bits_ref: (i, 0)),
                pl.BlockSpec((E, D_MODEL, OUT_DIM),
                             lambda i, bits_ref: (0, 0, 0)),
                pl.BlockSpec((E, OUT_DIM), lambda i, bits_ref: (0, 0)),
            ],
            out_specs=pl.BlockSpec(memory_space=pl.MemorySpace.ANY),
            scratch_shapes=[
                pltpu.VMEM((E, BN, OUT_DIM), jnp.float32),
                pltpu.VMEM((E, BN, OUT_DIM), jnp.float32),
                pltpu.SemaphoreType.DMA,
                pltpu.SemaphoreType.DMA,
            ],
        ),
        out_shape=jax.ShapeDtypeStruct((E, N, OUT_DIM), jnp.float32),
        compiler_params=pltpu.CompilerParams(
            dimension_semantics=("arbitrary",),
        ),
    )(bits, bidx, didx, emb, W, b)
    return out


# speedup vs baseline: 1.5285x; 1.0173x over previous
import jax
import jax.numpy as jnp
from jax import lax
from jax.experimental import pallas as pl
from jax.experimental.pallas import tpu as pltpu

N = 8192
D_MODEL = 1024
OUT_DIM = 256
E = 8
G = 64
BN = 512
NB = N // BN


def _moe_block_kernel(glo_ref, ghi_ref, didx_s_ref, bidx_ref, didx_ref,
                      emb_ref, W_ref, b_ref, out_hbm, y0, y1, sem0, sem1):
    i = pl.program_id(0)
    bits = lax.fori_loop(
        glo_ref[i], ghi_ref[i] + 1,
        lambda g, acc: acc | (1 << didx_s_ref[g]), 0,
        unroll=False)
    bidx = bidx_ref[0]
    g_iota = jax.lax.broadcasted_iota(jnp.int32, (BN, G), 1)
    onehot = bidx == g_iota
    didx = didx_ref[...]
    e_atom = jnp.sum(jnp.where(onehot, didx, 0), axis=1,
                     keepdims=True)
    x = emb_ref[...].astype(jnp.bfloat16)

    def run(y_ref, sem):
        @pl.when(i >= 2)
        def _():
            pltpu.make_async_copy(
                y_ref, out_hbm.at[:, pl.ds((i - 2) * BN, BN), :], sem
            ).wait()

        for e in range(E):
            present = ((bits >> e) & 1) == 1

            @pl.when(present)
            def _(e=e):
                mask = e_atom == e
                y = jnp.dot(x, W_ref[e].astype(jnp.bfloat16),
                            preferred_element_type=jnp.float32)
                y = y + b_ref[pl.ds(e, 1), :]
                y_ref[e] = jnp.where(mask, y, 0.0)

            @pl.when(jnp.logical_not(present))
            def _(e=e):
                y_ref[e] = jnp.zeros((BN, OUT_DIM), jnp.float32)

        pltpu.make_async_copy(
            y_ref, out_hbm.at[:, pl.ds(i * BN, BN), :], sem
        ).start()

    @pl.when(i % 2 == 0)
    def _():
        run(y0, sem0)

    @pl.when(i % 2 == 1)
    def _():
        run(y1, sem1)

    @pl.when(i == NB - 1)
    def _():
        pltpu.make_async_copy(
            y0, out_hbm.at[:, pl.ds((NB - 2) * BN, BN), :], sem0
        ).wait()
        pltpu.make_async_copy(
            y1, out_hbm.at[:, pl.ds((NB - 1) * BN, BN), :], sem1
        ).wait()


def kernel(emb, W, b, batch_idx, dataset_idx):
    bi = batch_idx.astype(jnp.int32)
    bidx = bi.reshape(NB, BN, 1)
    d32 = dataset_idx.astype(jnp.int32)
    didx = d32.reshape(1, G)
    br = bi.reshape(NB, BN)
    g_lo = br[:, 0]
    g_hi = br[:, BN - 1]

    out = pl.pallas_call(
        _moe_block_kernel,
        grid_spec=pltpu.PrefetchScalarGridSpec(
            num_scalar_prefetch=3,
            grid=(NB,),
            in_specs=[
                pl.BlockSpec((1, BN, 1), lambda i, *_: (i, 0, 0)),
                pl.BlockSpec((1, G), lambda i, *_: (0, 0)),
                pl.BlockSpec((BN, D_MODEL), lambda i, *_: (i, 0)),
                pl.BlockSpec((E, D_MODEL, OUT_DIM), lambda i, *_: (0, 0, 0)),
                pl.BlockSpec((E, OUT_DIM), lambda i, *_: (0, 0)),
            ],
            out_specs=pl.BlockSpec(memory_space=pl.MemorySpace.ANY),
            scratch_shapes=[
                pltpu.VMEM((E, BN, OUT_DIM), jnp.float32),
                pltpu.VMEM((E, BN, OUT_DIM), jnp.float32),
                pltpu.SemaphoreType.DMA,
                pltpu.SemaphoreType.DMA,
            ],
        ),
        out_shape=jax.ShapeDtypeStruct((E, N, OUT_DIM), jnp.float32),
        compiler_params=pltpu.CompilerParams(
            dimension_semantics=("arbitrary",),
        ),
    )(g_lo, g_hi, d32, bidx, didx, emb, W, b)
    return out
